# baseline (speedup 1.0000x reference)
import jax
import jax.numpy as jnp
from jax.experimental import pallas as pl
from jax.experimental.pallas import tpu as pltpu

_BK = 200


def _matmul_body(xt_ref, w_ref, o_ref):
    x = xt_ref[...].astype(jnp.bfloat16)
    w = w_ref[...].astype(jnp.bfloat16)
    acc = jax.lax.dot_general(
        x, w, (((0,), (0,)), ((), ())),
        preferred_element_type=jnp.float32)
    j = pl.program_id(0)

    @pl.when(j == 0)
    def _init():
        o_ref[...] = acc

    @pl.when(j != 0)
    def _accum():
        o_ref[...] += acc


def kernel(inputs, kernel):
    m, k = inputs.shape
    _, n = kernel.shape
    bk = _BK if k % _BK == 0 else k
    xt = inputs.T
    return pl.pallas_call(
        _matmul_body,
        grid=(k // bk,),
        in_specs=[
            pl.BlockSpec((bk, m), lambda j: (j, 0)),
            pl.BlockSpec((bk, n), lambda j: (j, 0)),
        ],
        out_specs=pl.BlockSpec((m, n), lambda j: (0, 0)),
        out_shape=jax.ShapeDtypeStruct((m, n), jnp.float32),
        compiler_params=pltpu.CompilerParams(
            dimension_semantics=("arbitrary",),
        ),
    )(xt, kernel)
